# baseline (speedup 1.0000x reference)
import math

import jax
import jax.numpy as jnp
from jax.experimental import pallas as pl
from jax.experimental.pallas import tpu as pltpu

_C = math.sqrt(2.0 / math.pi)
_K = _C * 0.044715

_D = 2048
_CH = 1024
_NBUF = 2


def _gelu(x):
    u = x * (_C + _K * (x * x))
    h = 0.5 * x
    return h + h * jnp.tanh(u)


def _pipeline_body(x_hbm, o_hbm, inb, outb, in_sem, out_sem):
    n_rows = x_hbm.shape[0]
    nchunks = n_rows // _CH
    rounds = nchunks // _NBUF

    def start_in(chunk, slot):
        pltpu.make_async_copy(
            x_hbm.at[pl.ds(chunk * _CH, _CH), :], inb.at[slot], in_sem.at[slot]
        ).start()

    def wait_in(chunk, slot):
        pltpu.make_async_copy(
            x_hbm.at[pl.ds(chunk * _CH, _CH), :], inb.at[slot], in_sem.at[slot]
        ).wait()

    def start_out(chunk, slot):
        pltpu.make_async_copy(
            outb.at[slot], o_hbm.at[pl.ds(chunk * _CH, _CH), :], out_sem.at[slot]
        ).start()

    def wait_out(chunk, slot):
        pltpu.make_async_copy(
            outb.at[slot], o_hbm.at[pl.ds(chunk * _CH, _CH), :], out_sem.at[slot]
        ).wait()

    for s in range(_NBUF):
        start_in(s, s)

    def round_body(r, _):
        for s in range(_NBUF):
            chunk = r * _NBUF + s
            wait_in(chunk, s)

            @pl.when(r > 0)
            def _():
                wait_out(chunk - _NBUF, s)

            outb[s] = _gelu(inb[s])
            start_out(chunk, s)

            @pl.when(r < rounds - 1)
            def _():
                start_in(chunk + _NBUF, s)

        return 0

    jax.lax.fori_loop(0, rounds, round_body, 0)

    for s in range(_NBUF):
        wait_out(nchunks - _NBUF + s, s)


def kernel(x, log_k_blend):
    B, T, D = x.shape
    R = B * T
    x2 = x.reshape(R, D)
    out = pl.pallas_call(
        _pipeline_body,
        in_specs=[pl.BlockSpec(memory_space=pltpu.HBM)],
        out_specs=pl.BlockSpec(memory_space=pltpu.HBM),
        out_shape=jax.ShapeDtypeStruct((R, D), x.dtype),
        scratch_shapes=[
            pltpu.VMEM((_NBUF, _CH, _D), jnp.float32),
            pltpu.VMEM((_NBUF, _CH, _D), jnp.float32),
            pltpu.SemaphoreType.DMA((_NBUF,)),
            pltpu.SemaphoreType.DMA((_NBUF,)),
        ],
    )(x2)
    return out.reshape(B, T, D)


# speedup vs baseline: 1.0530x; 1.0530x over previous
import math

import jax
import jax.numpy as jnp
from jax.experimental import pallas as pl
from jax.experimental.pallas import tpu as pltpu

_C = math.sqrt(2.0 / math.pi)
_K = _C * 0.044715

_D = 2048
_CH = 512
_NBUF = 4


def _gelu(x):
    u = x * (_C + _K * (x * x))
    h = 0.5 * x
    return h + h * jnp.tanh(u)


def _pipeline_body(x_hbm, o_hbm, inb, outb, in_sem, out_sem):
    n_rows = x_hbm.shape[0]
    nchunks = n_rows // _CH
    rounds = nchunks // _NBUF

    def start_in(chunk, slot):
        pltpu.make_async_copy(
            x_hbm.at[pl.ds(chunk * _CH, _CH), :], inb.at[slot], in_sem.at[slot]
        ).start()

    def wait_in(chunk, slot):
        pltpu.make_async_copy(
            x_hbm.at[pl.ds(chunk * _CH, _CH), :], inb.at[slot], in_sem.at[slot]
        ).wait()

    def start_out(chunk, slot):
        pltpu.make_async_copy(
            outb.at[slot], o_hbm.at[pl.ds(chunk * _CH, _CH), :], out_sem.at[slot]
        ).start()

    def wait_out(chunk, slot):
        pltpu.make_async_copy(
            outb.at[slot], o_hbm.at[pl.ds(chunk * _CH, _CH), :], out_sem.at[slot]
        ).wait()

    for s in range(_NBUF):
        start_in(s, s)

    def round_body(r, _):
        for s in range(_NBUF):
            chunk = r * _NBUF + s
            wait_in(chunk, s)

            @pl.when(r > 0)
            def _():
                wait_out(chunk - _NBUF, s)

            outb[s] = _gelu(inb[s])
            start_out(chunk, s)

            @pl.when(r < rounds - 1)
            def _():
                start_in(chunk + _NBUF, s)

        return 0

    jax.lax.fori_loop(0, rounds, round_body, 0)

    for s in range(_NBUF):
        wait_out(nchunks - _NBUF + s, s)


def kernel(x, log_k_blend):
    B, T, D = x.shape
    R = B * T
    x2 = x.reshape(R, D)
    out = pl.pallas_call(
        _pipeline_body,
        in_specs=[pl.BlockSpec(memory_space=pltpu.HBM)],
        out_specs=pl.BlockSpec(memory_space=pltpu.HBM),
        out_shape=jax.ShapeDtypeStruct((R, D), x.dtype),
        scratch_shapes=[
            pltpu.VMEM((_NBUF, _CH, _D), jnp.float32),
            pltpu.VMEM((_NBUF, _CH, _D), jnp.float32),
            pltpu.SemaphoreType.DMA((_NBUF,)),
            pltpu.SemaphoreType.DMA((_NBUF,)),
        ],
    )(x2)
    return out.reshape(B, T, D)
